# baseline (speedup 1.0000x reference)
import functools

import jax
import jax.numpy as jnp
from jax import lax
from jax.experimental import pallas as pl
from jax.experimental.pallas import tpu as pltpu
from jax.experimental.pallas import tpu_sc as plsc

L = 16
NC = 2
NS = 16
NW = NC * NS
CB = 128


def _interp_body(ntime, nbatch, chunk,
                 times_hbm, data_hbm, t_hbm, out_hbm,
                 colblk, t0v, tv, dblk, xblk, ov, cnt_smem, sem):
    cid = lax.axis_index("c")
    sid = lax.axis_index("s")
    wid = sid * NC + cid
    base = pl.multiple_of(wid * chunk, 128)
    rows_per_tile = ntime // NS

    row0 = pl.multiple_of(sid * rows_per_tile, 8)
    cp_t0 = pltpu.make_async_copy(t_hbm.at[pl.ds(0, L)], t0v, sem)
    cp_t = pltpu.make_async_copy(t_hbm.at[pl.ds(base, chunk)], tv, sem)
    half = rows_per_tile // 2
    row1 = pl.multiple_of(row0 + half, 8)
    cp_c = pltpu.make_async_copy(
        times_hbm.at[pl.ds(row0, half), pl.ds(0, CB)],
        colblk.at[pl.ds(0, half)], sem)
    cp_c2 = pltpu.make_async_copy(
        times_hbm.at[pl.ds(row1, half), pl.ds(0, CB)],
        colblk.at[pl.ds(half, half)], sem)
    cp_d = pltpu.make_async_copy(
        data_hbm.at[pl.ds(0, 8), pl.ds(base, chunk)], dblk.at[pl.ds(0, 8)], sem)
    cp_x = pltpu.make_async_copy(
        times_hbm.at[pl.ds(0, 8), pl.ds(base, chunk)], xblk.at[pl.ds(0, 8)], sem)
    cp_t0.start()
    cp_c.start()
    cp_c2.start()
    cp_t.start()
    cp_d.start()
    cp_x.start()
    cp_t0.wait()
    cp_c.wait()

    t0 = t0v[...][0]
    t0_vec = jnp.full((L,), t0, dtype=jnp.float32)

    @plsc.parallel_loop(0, half, unroll=8,
                        carry=jnp.zeros((L,), dtype=jnp.int32))
    def cnt0(r, c):
        vals = colblk[r, pl.ds(0, L)]
        return c + jnp.where(vals < t0_vec, 1, 0).astype(jnp.int32)

    cp_c2.wait()

    @plsc.parallel_loop(half, rows_per_tile, unroll=8, carry=cnt0)
    def cnt(r, c):
        vals = colblk[r, pl.ds(0, L)]
        return c + jnp.where(vals < t0_vec, 1, 0).astype(jnp.int32)

    @pl.when(sid == 0)
    def _zero():
        cnt_smem[0] = 0

    plsc.subcore_barrier()
    plsc.fetch_and_add(cnt_smem.at[0], cnt[0], subcore_id=0)

    cp_d.wait()
    cp_x.wait()
    cp_t.wait()

    def interp_step(lo, k, _):
        s = pl.ds(k * L, L)
        a = dblk[lo, s]
        b = dblk[lo + 1, s]
        p = xblk[lo, s]
        q = xblk[lo + 1, s]
        tt = tv[s]
        ov[s] = a + (b - a) / (q - p) * (tt - p)
        return 0

    lax.fori_loop(0, chunk // L, functools.partial(interp_step, 0), 0)

    cp_o = pltpu.make_async_copy(ov, out_hbm.at[pl.ds(base, chunk)], sem)
    cp_o.start()

    plsc.subcore_barrier()
    total = plsc.fetch_and_add(cnt_smem.at[0], 0, subcore_id=0)

    gi = jnp.where(total >= ntime, 1, jnp.maximum(total, 1)).astype(jnp.int32)
    gim1 = gi - 1

    rb = pl.multiple_of(
        jnp.minimum((gim1 // 8) * 8, ntime - 2 * 8).astype(jnp.int32), 8)
    loff = gim1 - rb

    @pl.when(gim1 != 0)
    def _redo():
        cp_d2 = pltpu.make_async_copy(
            data_hbm.at[pl.ds(rb, 2 * 8), pl.ds(base, chunk)], dblk, sem)
        cp_x2 = pltpu.make_async_copy(
            times_hbm.at[pl.ds(rb, 2 * 8), pl.ds(base, chunk)], xblk, sem)
        cp_d2.start()
        cp_x2.start()
        cp_d2.wait()
        cp_x2.wait()
        cp_o.wait()
        lax.fori_loop(0, chunk // L, functools.partial(interp_step, loff), 0)
        pltpu.sync_copy(ov, out_hbm.at[pl.ds(base, chunk)])

    @pl.when(gim1 == 0)
    def _commit():
        cp_o.wait()


def kernel(times, data, t):
    ntime, nbatch = times.shape
    chunk = nbatch // NW
    mesh = plsc.VectorSubcoreMesh(core_axis_name="c", subcore_axis_name="s")
    body = functools.partial(_interp_body, ntime, nbatch, chunk)
    run = pl.kernel(
        body,
        mesh=mesh,
        out_type=jax.ShapeDtypeStruct((nbatch,), jnp.float32),
        scratch_types=[
            pltpu.VMEM((ntime // NS, CB), jnp.float32),
            pltpu.VMEM((L,), jnp.float32),
            pltpu.VMEM((chunk,), jnp.float32),
            pltpu.VMEM((2 * 8, chunk), jnp.float32),
            pltpu.VMEM((2 * 8, chunk), jnp.float32),
            pltpu.VMEM((chunk,), jnp.float32),
            pltpu.SMEM((1,), jnp.int32),
            pltpu.SemaphoreType.DMA,
        ],
    )
    return run(times, data, t)


# speedup vs baseline: 1.0195x; 1.0195x over previous
import functools

import jax
import jax.numpy as jnp
from jax import lax
from jax.experimental import pallas as pl
from jax.experimental.pallas import tpu as pltpu
from jax.experimental.pallas import tpu_sc as plsc

L = 16
NC = 2
NS = 16
NW = NC * NS
CB = 128


def _interp_body(ntime, nbatch, chunk,
                 times_hbm, data_hbm, t_hbm, out_hbm,
                 colblk, t0v, tv, dblk, xblk, ov, cnt_smem, sem):
    cid = lax.axis_index("c")
    sid = lax.axis_index("s")
    wid = sid * NC + cid
    base = pl.multiple_of(wid * chunk, 128)
    rows_per_tile = ntime // NS

    row0 = pl.multiple_of(sid * rows_per_tile, 8)
    cp_t0 = pltpu.make_async_copy(t_hbm.at[pl.ds(0, L)], t0v, sem)
    cp_t = pltpu.make_async_copy(t_hbm.at[pl.ds(base, chunk)], tv, sem)
    half = rows_per_tile // 2
    row1 = pl.multiple_of(row0 + half, 8)
    cp_c = pltpu.make_async_copy(
        times_hbm.at[pl.ds(row0, half), pl.ds(0, CB)],
        colblk.at[pl.ds(0, half)], sem)
    cp_c2 = pltpu.make_async_copy(
        times_hbm.at[pl.ds(row1, half), pl.ds(0, CB)],
        colblk.at[pl.ds(half, half)], sem)
    cp_d = pltpu.make_async_copy(
        data_hbm.at[pl.ds(0, 8), pl.ds(base, chunk)], dblk.at[pl.ds(0, 8)], sem)
    cp_x = pltpu.make_async_copy(
        times_hbm.at[pl.ds(0, 8), pl.ds(base, chunk)], xblk.at[pl.ds(0, 8)], sem)
    cp_t0.start()
    cp_c.start()
    cp_c2.start()
    cp_t.start()
    cp_d.start()
    cp_x.start()
    cp_t0.wait()
    cp_c.wait()

    t0 = t0v[...][0]
    t0_vec = jnp.full((L,), t0, dtype=jnp.float32)

    @plsc.parallel_loop(0, half, unroll=8,
                        carry=jnp.zeros((L,), dtype=jnp.int32))
    def cnt0(r, c):
        vals = colblk[r, pl.ds(0, L)]
        return c + jnp.where(vals < t0_vec, 1, 0).astype(jnp.int32)

    cp_c2.wait()

    @plsc.parallel_loop(half, rows_per_tile, unroll=8, carry=cnt0)
    def cnt(r, c):
        vals = colblk[r, pl.ds(0, L)]
        return c + jnp.where(vals < t0_vec, 1, 0).astype(jnp.int32)

    @pl.when(sid == 0)
    def _zero():
        cnt_smem[0] = 0

    plsc.subcore_barrier()
    plsc.fetch_and_add(cnt_smem.at[0], cnt[0], subcore_id=0)

    cp_d.wait()
    cp_x.wait()
    cp_t.wait()

    def interp_step(lo, k):
        s = pl.ds(k * L, L)
        a = dblk[lo, s]
        b = dblk[lo + 1, s]
        p = xblk[lo, s]
        q = xblk[lo + 1, s]
        tt = tv[s]
        ov[s] = a + (b - a) / (q - p) * (tt - p)

    plsc.parallel_loop(0, chunk // L, unroll=4)(
        functools.partial(interp_step, 0))

    cp_o = pltpu.make_async_copy(ov, out_hbm.at[pl.ds(base, chunk)], sem)
    cp_o.start()

    plsc.subcore_barrier()
    total = plsc.fetch_and_add(cnt_smem.at[0], 0, subcore_id=0)

    gi = jnp.where(total >= ntime, 1, jnp.maximum(total, 1)).astype(jnp.int32)
    gim1 = gi - 1

    rb = pl.multiple_of(
        jnp.minimum((gim1 // 8) * 8, ntime - 2 * 8).astype(jnp.int32), 8)
    loff = gim1 - rb

    @pl.when(gim1 != 0)
    def _redo():
        cp_d2 = pltpu.make_async_copy(
            data_hbm.at[pl.ds(rb, 2 * 8), pl.ds(base, chunk)], dblk, sem)
        cp_x2 = pltpu.make_async_copy(
            times_hbm.at[pl.ds(rb, 2 * 8), pl.ds(base, chunk)], xblk, sem)
        cp_d2.start()
        cp_x2.start()
        cp_d2.wait()
        cp_x2.wait()
        cp_o.wait()
        plsc.parallel_loop(0, chunk // L, unroll=4)(
            functools.partial(interp_step, loff))
        pltpu.sync_copy(ov, out_hbm.at[pl.ds(base, chunk)])

    @pl.when(gim1 == 0)
    def _commit():
        cp_o.wait()


def kernel(times, data, t):
    ntime, nbatch = times.shape
    chunk = nbatch // NW
    mesh = plsc.VectorSubcoreMesh(core_axis_name="c", subcore_axis_name="s")
    body = functools.partial(_interp_body, ntime, nbatch, chunk)
    run = pl.kernel(
        body,
        mesh=mesh,
        out_type=jax.ShapeDtypeStruct((nbatch,), jnp.float32),
        scratch_types=[
            pltpu.VMEM((ntime // NS, CB), jnp.float32),
            pltpu.VMEM((L,), jnp.float32),
            pltpu.VMEM((chunk,), jnp.float32),
            pltpu.VMEM((2 * 8, chunk), jnp.float32),
            pltpu.VMEM((2 * 8, chunk), jnp.float32),
            pltpu.VMEM((chunk,), jnp.float32),
            pltpu.SMEM((1,), jnp.int32),
            pltpu.SemaphoreType.DMA,
        ],
    )
    return run(times, data, t)


# speedup vs baseline: 1.0223x; 1.0028x over previous
import functools

import jax
import jax.numpy as jnp
from jax import lax
from jax.experimental import pallas as pl
from jax.experimental.pallas import tpu as pltpu
from jax.experimental.pallas import tpu_sc as plsc

L = 16
NC = 2
NS = 16
NW = NC * NS
CB = 128


def _interp_body(ntime, nbatch, chunk,
                 times_hbm, data_hbm, t_hbm, out_hbm,
                 colblk, t0v, tv, dblk, xblk, ov, cnt_smem, sem):
    cid = lax.axis_index("c")
    sid = lax.axis_index("s")
    wid = sid * NC + cid
    base = pl.multiple_of(wid * chunk, 128)
    rows_per_tile = ntime // NS

    row0 = pl.multiple_of(sid * rows_per_tile, 8)
    cp_t0 = pltpu.make_async_copy(t_hbm.at[pl.ds(0, L)], t0v, sem)
    cp_t = pltpu.make_async_copy(t_hbm.at[pl.ds(base, chunk)], tv, sem)
    half = rows_per_tile // 2
    row1 = pl.multiple_of(row0 + half, 8)
    cp_c = pltpu.make_async_copy(
        times_hbm.at[pl.ds(row0, half), pl.ds(0, CB)],
        colblk.at[pl.ds(0, half)], sem)
    cp_c2 = pltpu.make_async_copy(
        times_hbm.at[pl.ds(row1, half), pl.ds(0, CB)],
        colblk.at[pl.ds(half, half)], sem)
    cp_d = pltpu.make_async_copy(
        data_hbm.at[pl.ds(0, 8), pl.ds(base, chunk)], dblk.at[pl.ds(0, 8)], sem)
    cp_x = pltpu.make_async_copy(
        times_hbm.at[pl.ds(0, 8), pl.ds(base, chunk)], xblk.at[pl.ds(0, 8)], sem)
    cp_t0.start()
    cp_c.start()
    cp_c2.start()
    cp_t.start()
    cp_d.start()
    cp_x.start()

    @pl.when(sid == 0)
    def _zero():
        cnt_smem[0] = 0

    plsc.subcore_barrier()

    cp_t0.wait()
    cp_c.wait()

    t0 = t0v[...][0]
    t0_vec = jnp.full((L,), t0, dtype=jnp.float32)

    @plsc.parallel_loop(0, half, unroll=8,
                        carry=jnp.zeros((L,), dtype=jnp.int32))
    def cnt0(r, c):
        vals = colblk[r, pl.ds(0, L)]
        return c + jnp.where(vals < t0_vec, 1, 0).astype(jnp.int32)

    cp_c2.wait()

    @plsc.parallel_loop(half, rows_per_tile, unroll=8, carry=cnt0)
    def cnt(r, c):
        vals = colblk[r, pl.ds(0, L)]
        return c + jnp.where(vals < t0_vec, 1, 0).astype(jnp.int32)

    plsc.fetch_and_add(cnt_smem.at[0], cnt[0], subcore_id=0)

    cp_d.wait()
    cp_x.wait()
    cp_t.wait()

    def interp_step(lo, k):
        s = pl.ds(k * L, L)
        a = dblk[lo, s]
        b = dblk[lo + 1, s]
        p = xblk[lo, s]
        q = xblk[lo + 1, s]
        tt = tv[s]
        ov[s] = a + (b - a) / (q - p) * (tt - p)

    plsc.parallel_loop(0, chunk // L, unroll=4)(
        functools.partial(interp_step, 0))

    cp_o = pltpu.make_async_copy(ov, out_hbm.at[pl.ds(base, chunk)], sem)
    cp_o.start()

    plsc.subcore_barrier()
    total = plsc.fetch_and_add(cnt_smem.at[0], 0, subcore_id=0)

    gi = jnp.where(total >= ntime, 1, jnp.maximum(total, 1)).astype(jnp.int32)
    gim1 = gi - 1

    rb = pl.multiple_of(
        jnp.minimum((gim1 // 8) * 8, ntime - 2 * 8).astype(jnp.int32), 8)
    loff = gim1 - rb

    @pl.when(gim1 != 0)
    def _redo():
        cp_d2 = pltpu.make_async_copy(
            data_hbm.at[pl.ds(rb, 2 * 8), pl.ds(base, chunk)], dblk, sem)
        cp_x2 = pltpu.make_async_copy(
            times_hbm.at[pl.ds(rb, 2 * 8), pl.ds(base, chunk)], xblk, sem)
        cp_d2.start()
        cp_x2.start()
        cp_d2.wait()
        cp_x2.wait()
        cp_o.wait()
        plsc.parallel_loop(0, chunk // L, unroll=4)(
            functools.partial(interp_step, loff))
        pltpu.sync_copy(ov, out_hbm.at[pl.ds(base, chunk)])

    @pl.when(gim1 == 0)
    def _commit():
        cp_o.wait()


def kernel(times, data, t):
    ntime, nbatch = times.shape
    chunk = nbatch // NW
    mesh = plsc.VectorSubcoreMesh(core_axis_name="c", subcore_axis_name="s")
    body = functools.partial(_interp_body, ntime, nbatch, chunk)
    run = pl.kernel(
        body,
        mesh=mesh,
        out_type=jax.ShapeDtypeStruct((nbatch,), jnp.float32),
        scratch_types=[
            pltpu.VMEM((ntime // NS, CB), jnp.float32),
            pltpu.VMEM((L,), jnp.float32),
            pltpu.VMEM((chunk,), jnp.float32),
            pltpu.VMEM((2 * 8, chunk), jnp.float32),
            pltpu.VMEM((2 * 8, chunk), jnp.float32),
            pltpu.VMEM((chunk,), jnp.float32),
            pltpu.SMEM((1,), jnp.int32),
            pltpu.SemaphoreType.DMA,
        ],
    )
    return run(times, data, t)
